# baseline (speedup 1.0000x reference)
import jax
import jax.numpy as jnp
from jax import lax
from jax.experimental import pallas as pl
from jax.experimental.pallas import tpu as pltpu
from jax.experimental.pallas import tpu_sc as plsc

N_NODES = 10000
N_EDGES = 160000
IN_CH = 256
HID_CH = 512
OUT_CH = 256

NC = 2
NS = 16
CHUNK = 128
E_PAD = 163840
N_PAD = 10240
RPT = N_PAD // NS
HALF = IN_CH // 2
RBLK = 2000
GRID = N_NODES // RBLK

def _mesh():
    return plsc.VectorSubcoreMesh(core_axis_name="c", subcore_axis_name="s")


def _deg_body(dstw, onesrow, zrows, deg_out, dst_idx, ones_v, dacc, dsem):
    c = lax.axis_index("c")
    s = lax.axis_index("s")
    w = c * NS + s
    n_chunks = dstw.shape[1]
    pltpu.sync_copy(zrows, dacc.at[pl.ds(s * RPT, RPT)])
    pltpu.sync_copy(onesrow, ones_v)
    pltpu.sync_copy(dstw.at[w], dst_idx)
    plsc.subcore_barrier()

    def step(j, carry):
        pltpu.async_copy(ones_v, dacc.at[dst_idx.at[j]], dsem, add=True)
        return carry

    lax.fori_loop(0, n_chunks, step, 0)

    def drain(j, carry):
        pltpu.make_async_copy(ones_v, dacc.at[dst_idx.at[0]], dsem).wait()
        return carry

    lax.fori_loop(0, n_chunks, drain, 0)
    plsc.subcore_barrier()
    pltpu.sync_copy(dacc.at[pl.ds(s * RPT, RPT)],
                    deg_out.at[c, pl.ds(s * RPT, RPT)])


def _deg_kernel(dstw, onesrow, zrows):
    n_chunks = dstw.shape[1]
    return pl.kernel(
        _deg_body,
        out_type=jax.ShapeDtypeStruct((NC, N_PAD, HALF), jnp.float32),
        mesh=_mesh(),
        scratch_types=[
            pltpu.VMEM((n_chunks, CHUNK), jnp.int32),
            pltpu.VMEM((CHUNK, HALF), jnp.float32),
            pltpu.VMEM_SHARED((N_PAD, HALF), jnp.float32),
            pltpu.SemaphoreType.DMA,
        ],
    )(dstw, onesrow, zrows)


ACHUNK = 128
NBUF = 2


def _agg_body(table, srcg, dstg, zrows, sb_out, src_idx, *rest):
    rows = rest[:NBUF]
    dring = rest[NBUF:2 * NBUF]
    acc = rest[2 * NBUF]
    gs = rest[2 * NBUF + 1:2 * NBUF + 1 + NBUF]
    ds = rest[2 * NBUF + 1 + NBUF:2 * NBUF + 1 + 2 * NBUF]
    ss = rest[2 * NBUF + 1 + 2 * NBUF:]
    c = lax.axis_index("c")
    s = lax.axis_index("s")
    w = c * NS + s
    n_chunks = dstg.shape[1]
    pltpu.sync_copy(zrows, acc.at[pl.ds(s * RPT, RPT)])
    pltpu.sync_copy(srcg.at[w], src_idx)
    plsc.subcore_barrier()

    def gather(j, b):
        pltpu.async_copy(table.at[src_idx.at[pl.ds(j * ACHUNK, ACHUNK)]],
                         rows[b], gs[b])
        pltpu.async_copy(dstg.at[s, j], dring[b], ds[b])

    def wait_gather(j, b):
        pltpu.make_async_copy(table.at[src_idx.at[pl.ds(j * ACHUNK, ACHUNK)]],
                              rows[b], gs[b]).wait()
        pltpu.make_async_copy(dstg.at[s, j], dring[b], ds[b]).wait()

    def scatter(j, b):
        pltpu.async_copy(rows[b], acc.at[dring[b]], ss[b], add=True)

    def wait_scatter(j, b):
        pltpu.make_async_copy(rows[b], acc.at[dring[b]], ss[b]).wait()

    gather(0, 0)
    gather(1, 1)

    def step(k, carry):
        for b in range(NBUF):
            j = k * NBUF + b
            wait_gather(j, b)
            scatter(j, b)
            wait_scatter(j, b)
            gather(j + NBUF, b)
        return carry

    lax.fori_loop(0, n_chunks // NBUF - 1, step, 0)
    for b in range(NBUF):
        j = n_chunks - NBUF + b
        wait_gather(j, b)
        scatter(j, b)
        wait_scatter(j, b)

    plsc.subcore_barrier()
    pltpu.sync_copy(acc.at[pl.ds(s * RPT, RPT)],
                    sb_out.at[c, pl.ds(s * RPT, RPT)])


def _agg_kernel(table, srcg, dstg, zrows):
    e_per_tile = srcg.shape[1]
    return pl.kernel(
        _agg_body,
        out_type=jax.ShapeDtypeStruct((NC, N_PAD, HALF), jnp.float32),
        mesh=_mesh(),
        scratch_types=[
            pltpu.VMEM((e_per_tile,), jnp.int32),
        ] + [pltpu.VMEM((ACHUNK, HALF), jnp.float32)] * NBUF
          + [pltpu.VMEM((ACHUNK,), jnp.int32)] * NBUF + [
            pltpu.VMEM_SHARED((N_PAD, HALF), jnp.float32),
        ] + [pltpu.SemaphoreType.DMA] * (3 * NBUF),
    )(table, srcg, dstg, zrows)


def _dinv_of(dp):
    deg = dp[0, :, 0:1] + dp[1, :, 0:1] + 1.0
    return lax.rsqrt(deg)


def _scale_body(dp_ref, x_ref, out_ref):
    u = x_ref[...] * _dinv_of(dp_ref[...])
    out_ref[0] = u[:, :HALF]
    out_ref[1] = u[:, HALF:]


def _scale_kernel(deg_parts, x):
    return pl.pallas_call(
        _scale_body,
        grid=(GRID,),
        in_specs=[
            pl.BlockSpec((NC, RBLK, HALF), lambda i: (0, i, 0)),
            pl.BlockSpec((RBLK, IN_CH), lambda i: (i, 0)),
        ],
        out_specs=pl.BlockSpec((NC, RBLK, HALF), lambda i: (0, i, 0)),
        out_shape=jax.ShapeDtypeStruct((NC, N_NODES, HALF), jnp.float32),
    )(deg_parts, x)


def _mm_body(dp_ref, sb_ref, ub_ref, w1_ref, b1_ref, w2_ref, out_ref):
    dinv = _dinv_of(dp_ref[...])
    z = jnp.concatenate(
        [sb_ref[0] + ub_ref[0], sb_ref[1] + ub_ref[1]], axis=1) * dinv
    h = jnp.dot(z.astype(jnp.bfloat16), w1_ref[...],
                preferred_element_type=jnp.float32)
    h = jnp.maximum(h + b1_ref[...], 0.0)
    t = jnp.dot(h.astype(jnp.bfloat16), w2_ref[...],
                preferred_element_type=jnp.float32)
    u2 = t * dinv
    out_ref[0] = u2[:, :HALF]
    out_ref[1] = u2[:, HALF:]


def _mm_kernel(deg_parts, s1, u1b, W1, b1, W2):
    return pl.pallas_call(
        _mm_body,
        grid=(GRID,),
        in_specs=[
            pl.BlockSpec((NC, RBLK, HALF), lambda i: (0, i, 0)),
            pl.BlockSpec((NC, RBLK, HALF), lambda i: (0, i, 0)),
            pl.BlockSpec((NC, RBLK, HALF), lambda i: (0, i, 0)),
            pl.BlockSpec((IN_CH, HID_CH), lambda i: (0, 0)),
            pl.BlockSpec((1, HID_CH), lambda i: (0, 0)),
            pl.BlockSpec((HID_CH, OUT_CH), lambda i: (0, 0)),
        ],
        out_specs=pl.BlockSpec((NC, RBLK, HALF), lambda i: (0, i, 0)),
        out_shape=jax.ShapeDtypeStruct((NC, N_NODES, HALF), jnp.float32),
    )(deg_parts, s1, u1b, W1, b1, W2)


def _out_body(dp_ref, sb_ref, ub_ref, b2_ref, out_ref):
    dinv = _dinv_of(dp_ref[...])
    o = jnp.concatenate(
        [sb_ref[0] + ub_ref[0], sb_ref[1] + ub_ref[1]], axis=1) * dinv
    o = o + b2_ref[...]
    m = jnp.max(o, axis=1, keepdims=True)
    e = jnp.exp(o - m)
    lse = jnp.log(jnp.sum(e, axis=1, keepdims=True))
    out_ref[...] = o - m - lse


def _out_kernel(deg_parts, s2, u2b, b2):
    return pl.pallas_call(
        _out_body,
        grid=(GRID,),
        in_specs=[
            pl.BlockSpec((NC, RBLK, HALF), lambda i: (0, i, 0)),
            pl.BlockSpec((NC, RBLK, HALF), lambda i: (0, i, 0)),
            pl.BlockSpec((NC, RBLK, HALF), lambda i: (0, i, 0)),
            pl.BlockSpec((1, OUT_CH), lambda i: (0, 0)),
        ],
        out_specs=pl.BlockSpec((RBLK, OUT_CH), lambda i: (i, 0)),
        out_shape=jax.ShapeDtypeStruct((N_NODES, OUT_CH), jnp.float32),
    )(deg_parts, s2, u2b, b2)


def kernel(x, edge_index, W1, b1, W2, b2):
    src = edge_index[0].astype(jnp.int32)
    dst = edge_index[1].astype(jnp.int32)
    pad = E_PAD - N_EDGES
    src_p = jnp.concatenate([src, jnp.zeros((pad,), jnp.int32)])
    dst_p = jnp.concatenate([dst, jnp.full((pad,), N_NODES, jnp.int32)])
    dstw = dst_p.reshape(NC * NS, E_PAD // (NC * NS * CHUNK), CHUNK)
    src_t = src_p.reshape(NS, E_PAD // NS)
    srcg = jnp.concatenate([src_t, src_t + N_NODES], axis=0)
    dstg = dst_p.reshape(NS, E_PAD // (NS * ACHUNK), ACHUNK)
    onesrow = jnp.zeros((CHUNK, HALF), jnp.float32).at[:, 0].set(1.0)
    zrows = jnp.zeros((RPT, HALF), jnp.float32)

    deg_parts = _deg_kernel(dstw, onesrow, zrows)
    u1b = _scale_kernel(deg_parts, x)
    s1 = _agg_kernel(u1b.reshape(NC * N_NODES, HALF), srcg, dstg, zrows)
    u2b = _mm_kernel(deg_parts, s1, u1b, W1.astype(jnp.bfloat16),
                     b1.reshape(1, HID_CH), W2.astype(jnp.bfloat16))
    s2 = _agg_kernel(u2b.reshape(NC * N_NODES, HALF), srcg, dstg, zrows)
    return _out_kernel(deg_parts, s2, u2b, b2.reshape(1, OUT_CH))


# speedup vs baseline: 1.0448x; 1.0448x over previous
import jax
import jax.numpy as jnp
from jax import lax
from jax.experimental import pallas as pl
from jax.experimental.pallas import tpu as pltpu
from jax.experimental.pallas import tpu_sc as plsc

N_NODES = 10000
N_EDGES = 160000
IN_CH = 256
HID_CH = 512
OUT_CH = 256

NC = 2
NS = 16
CHUNK = 128
E_PAD = 163840
N_PAD = 10240
RPT = N_PAD // NS
HALF = IN_CH // 2
RBLK = 2000
GRID = N_NODES // RBLK

def _mesh():
    return plsc.VectorSubcoreMesh(core_axis_name="c", subcore_axis_name="s")


def _deg_body(dstw, onesrow, zrows, deg_out, dst_idx, ones_v, dacc, dsem):
    c = lax.axis_index("c")
    s = lax.axis_index("s")
    w = c * NS + s
    n_chunks = dstw.shape[1]
    pltpu.sync_copy(zrows, dacc.at[pl.ds(s * RPT, RPT)])
    pltpu.sync_copy(onesrow, ones_v)
    pltpu.sync_copy(dstw.at[w], dst_idx)
    plsc.subcore_barrier()

    def step(j, carry):
        pltpu.async_copy(ones_v, dacc.at[dst_idx.at[j]], dsem, add=True)
        return carry

    lax.fori_loop(0, n_chunks, step, 0)

    def drain(j, carry):
        pltpu.make_async_copy(ones_v, dacc.at[dst_idx.at[0]], dsem).wait()
        return carry

    lax.fori_loop(0, n_chunks, drain, 0)
    plsc.subcore_barrier()
    pltpu.sync_copy(dacc.at[pl.ds(s * RPT, RPT)],
                    deg_out.at[c, pl.ds(s * RPT, RPT)])


def _deg_kernel(dstw, onesrow, zrows):
    n_chunks = dstw.shape[1]
    return pl.kernel(
        _deg_body,
        out_type=jax.ShapeDtypeStruct((NC, N_PAD, HALF), jnp.float32),
        mesh=_mesh(),
        scratch_types=[
            pltpu.VMEM((n_chunks, CHUNK), jnp.int32),
            pltpu.VMEM((CHUNK, HALF), jnp.float32),
            pltpu.VMEM_SHARED((N_PAD, HALF), jnp.float32),
            pltpu.SemaphoreType.DMA,
        ],
    )(dstw, onesrow, zrows)


ACHUNK = 80
NBUF = 2


def _agg_body(table, srcg, dstg, zrows, sb_out, src_idx, dst_idx, *rest):
    rows = rest[:NBUF]
    acc = rest[NBUF]
    gs = rest[NBUF + 1:NBUF + 1 + NBUF]
    ss = rest[NBUF + 1 + NBUF:]
    c = lax.axis_index("c")
    s = lax.axis_index("s")
    w = c * NS + s
    n_chunks = dstg.shape[1]
    pltpu.sync_copy(zrows, acc.at[pl.ds(s * RPT, RPT)])
    pltpu.sync_copy(srcg.at[w], src_idx)
    pltpu.sync_copy(dstg.at[s], dst_idx)
    plsc.subcore_barrier()

    def gather(j, b):
        pltpu.async_copy(table.at[src_idx.at[pl.ds(j * ACHUNK, ACHUNK)]],
                         rows[b], gs[b])

    def wait_gather(j, b):
        pltpu.make_async_copy(table.at[src_idx.at[pl.ds(j * ACHUNK, ACHUNK)]],
                              rows[b], gs[b]).wait()

    def scatter(j, b):
        pltpu.async_copy(rows[b], acc.at[dst_idx.at[j]], ss[b], add=True)

    def wait_scatter(j, b):
        pltpu.make_async_copy(rows[b], acc.at[dst_idx.at[j]], ss[b]).wait()

    gather(0, 0)
    gather(1, 1)

    def step(k, carry):
        for b in range(NBUF):
            j = k * NBUF + b
            wait_gather(j, b)
            scatter(j, b)
            wait_scatter(j, b)
            gather(j + NBUF, b)
        return carry

    lax.fori_loop(0, n_chunks // NBUF - 1, step, 0)
    for b in range(NBUF):
        j = n_chunks - NBUF + b
        wait_gather(j, b)
        scatter(j, b)
        wait_scatter(j, b)

    plsc.subcore_barrier()
    pltpu.sync_copy(acc.at[pl.ds(s * RPT, RPT)],
                    sb_out.at[c, pl.ds(s * RPT, RPT)])


def _agg_kernel(table, srcg, dstg, zrows):
    e_per_tile = srcg.shape[1]
    n_chunks = e_per_tile // ACHUNK
    return pl.kernel(
        _agg_body,
        out_type=jax.ShapeDtypeStruct((NC, N_PAD, HALF), jnp.float32),
        mesh=_mesh(),
        scratch_types=[
            pltpu.VMEM((e_per_tile,), jnp.int32),
            pltpu.VMEM((n_chunks, ACHUNK), jnp.int32),
        ] + [pltpu.VMEM((ACHUNK, HALF), jnp.float32)] * NBUF + [
            pltpu.VMEM_SHARED((N_PAD, HALF), jnp.float32),
        ] + [pltpu.SemaphoreType.DMA] * (2 * NBUF),
    )(table, srcg, dstg, zrows)


def _dinv_of(dp):
    deg = dp[0, :, 0:1] + dp[1, :, 0:1] + 1.0
    return lax.rsqrt(deg)


def _scale_body(dp_ref, x_ref, out_ref):
    u = x_ref[...] * _dinv_of(dp_ref[...])
    out_ref[0] = u[:, :HALF]
    out_ref[1] = u[:, HALF:]


def _scale_kernel(deg_parts, x):
    return pl.pallas_call(
        _scale_body,
        grid=(GRID,),
        in_specs=[
            pl.BlockSpec((NC, RBLK, HALF), lambda i: (0, i, 0)),
            pl.BlockSpec((RBLK, IN_CH), lambda i: (i, 0)),
        ],
        out_specs=pl.BlockSpec((NC, RBLK, HALF), lambda i: (0, i, 0)),
        out_shape=jax.ShapeDtypeStruct((NC, N_NODES, HALF), jnp.float32),
    )(deg_parts, x)


def _mm_body(dp_ref, sb_ref, ub_ref, w1_ref, b1_ref, w2_ref, out_ref):
    dinv = _dinv_of(dp_ref[...])
    z = jnp.concatenate(
        [sb_ref[0] + ub_ref[0], sb_ref[1] + ub_ref[1]], axis=1) * dinv
    h = jnp.dot(z.astype(jnp.bfloat16), w1_ref[...],
                preferred_element_type=jnp.float32)
    h = jnp.maximum(h + b1_ref[...], 0.0)
    t = jnp.dot(h.astype(jnp.bfloat16), w2_ref[...],
                preferred_element_type=jnp.float32)
    u2 = t * dinv
    out_ref[0] = u2[:, :HALF]
    out_ref[1] = u2[:, HALF:]


def _mm_kernel(deg_parts, s1, u1b, W1, b1, W2):
    return pl.pallas_call(
        _mm_body,
        grid=(GRID,),
        in_specs=[
            pl.BlockSpec((NC, RBLK, HALF), lambda i: (0, i, 0)),
            pl.BlockSpec((NC, RBLK, HALF), lambda i: (0, i, 0)),
            pl.BlockSpec((NC, RBLK, HALF), lambda i: (0, i, 0)),
            pl.BlockSpec((IN_CH, HID_CH), lambda i: (0, 0)),
            pl.BlockSpec((1, HID_CH), lambda i: (0, 0)),
            pl.BlockSpec((HID_CH, OUT_CH), lambda i: (0, 0)),
        ],
        out_specs=pl.BlockSpec((NC, RBLK, HALF), lambda i: (0, i, 0)),
        out_shape=jax.ShapeDtypeStruct((NC, N_NODES, HALF), jnp.float32),
    )(deg_parts, s1, u1b, W1, b1, W2)


def _out_body(dp_ref, sb_ref, ub_ref, b2_ref, out_ref):
    dinv = _dinv_of(dp_ref[...])
    o = jnp.concatenate(
        [sb_ref[0] + ub_ref[0], sb_ref[1] + ub_ref[1]], axis=1) * dinv
    o = o + b2_ref[...]
    m = jnp.max(o, axis=1, keepdims=True)
    e = jnp.exp(o - m)
    lse = jnp.log(jnp.sum(e, axis=1, keepdims=True))
    out_ref[...] = o - m - lse


def _out_kernel(deg_parts, s2, u2b, b2):
    return pl.pallas_call(
        _out_body,
        grid=(GRID,),
        in_specs=[
            pl.BlockSpec((NC, RBLK, HALF), lambda i: (0, i, 0)),
            pl.BlockSpec((NC, RBLK, HALF), lambda i: (0, i, 0)),
            pl.BlockSpec((NC, RBLK, HALF), lambda i: (0, i, 0)),
            pl.BlockSpec((1, OUT_CH), lambda i: (0, 0)),
        ],
        out_specs=pl.BlockSpec((RBLK, OUT_CH), lambda i: (i, 0)),
        out_shape=jax.ShapeDtypeStruct((N_NODES, OUT_CH), jnp.float32),
    )(deg_parts, s2, u2b, b2)


def kernel(x, edge_index, W1, b1, W2, b2):
    src = edge_index[0].astype(jnp.int32)
    dst = edge_index[1].astype(jnp.int32)
    pad = E_PAD - N_EDGES
    src_p = jnp.concatenate([src, jnp.zeros((pad,), jnp.int32)])
    dst_p = jnp.concatenate([dst, jnp.full((pad,), N_NODES, jnp.int32)])
    dstw = dst_p.reshape(NC * NS, E_PAD // (NC * NS * CHUNK), CHUNK)
    src_t = src_p.reshape(NS, E_PAD // NS)
    srcg = jnp.concatenate([src_t, src_t + N_NODES], axis=0)
    dstg = dst_p.reshape(NS, E_PAD // (NS * ACHUNK), ACHUNK)
    onesrow = jnp.zeros((CHUNK, HALF), jnp.float32).at[:, 0].set(1.0)
    zrows = jnp.zeros((RPT, HALF), jnp.float32)

    deg_parts = _deg_kernel(dstw, onesrow, zrows)
    u1b = _scale_kernel(deg_parts, x)
    s1 = _agg_kernel(u1b.reshape(NC * N_NODES, HALF), srcg, dstg, zrows)
    u2b = _mm_kernel(deg_parts, s1, u1b, W1.astype(jnp.bfloat16),
                     b1.reshape(1, HID_CH), W2.astype(jnp.bfloat16))
    s2 = _agg_kernel(u2b.reshape(NC * N_NODES, HALF), srcg, dstg, zrows)
    return _out_kernel(deg_parts, s2, u2b, b2.reshape(1, OUT_CH))
